# baseline (speedup 1.0000x reference)
import functools

import jax
import jax.numpy as jnp
from jax import lax
from jax.experimental import pallas as pl
from jax.experimental.pallas import tpu as pltpu
from jax.experimental.pallas import tpu_sc as plsc

V = 1000000
D = 32
B = 16384

NC = 2
NS = 16
L = 16
NW = NC * NS
BPW = B // NW
CHUNK = 128
NCHUNK = BPW // CHUNK
NBLK = BPW // L

_mesh = plsc.VectorSubcoreMesh(core_axis_name="c", subcore_axis_name="s")


@functools.partial(
    pl.kernel,
    mesh=_mesh,
    out_type=(
        jax.ShapeDtypeStruct((B,), jnp.float32),
        jax.ShapeDtypeStruct((B,), jnp.float32),
    ),
    scratch_types=[
        pltpu.VMEM((BPW,), jnp.int32),
        pltpu.VMEM((BPW,), jnp.int32),
        pltpu.VMEM((BPW,), jnp.int32),
        pltpu.VMEM((D, BPW), jnp.float32),
        pltpu.VMEM((D, BPW), jnp.float32),
        pltpu.VMEM((D, BPW), jnp.float32),
        pltpu.VMEM((BPW,), jnp.float32),
        pltpu.VMEM((BPW,), jnp.float32),
        pltpu.SemaphoreType.DMA,
    ],
    compiler_params=pltpu.CompilerParams(
        needs_layout_passes=False, use_tc_tiling_on_sc=False
    ),
)
def _sc_dots(iword, owords, nwords, emb_iT, emb_oT, od_hbm, nd_hbm,
             iidx, oidx, nidx, ivecT, ovecT, nvecT, od_v, nd_v, sem):
    wid = lax.axis_index("s") * NC + lax.axis_index("c")
    base = wid * BPW

    pltpu.sync_copy(iword.at[pl.ds(base, BPW)], iidx)
    pltpu.sync_copy(owords.at[pl.ds(base, BPW)], oidx)
    pltpu.sync_copy(nwords.at[pl.ds(base, BPW)], nidx)

    def gather_d(d, carry):
        copies = []
        for c in range(NCHUNK):
            sl = pl.ds(c * CHUNK, CHUNK)
            copies.append(pltpu.async_copy(
                emb_iT.at[d].at[iidx.at[sl]], ivecT.at[d, sl], sem))
            copies.append(pltpu.async_copy(
                emb_oT.at[d].at[oidx.at[sl]], ovecT.at[d, sl], sem))
            copies.append(pltpu.async_copy(
                emb_oT.at[d].at[nidx.at[sl]], nvecT.at[d, sl], sem))
        for cp in copies:
            cp.wait()
        return carry

    lax.fori_loop(0, D, gather_d, 0)

    def blk_body(b, carry):
        sl = pl.ds(b * L, L)
        acc_o = jnp.zeros((L,), jnp.float32)
        acc_n = jnp.zeros((L,), jnp.float32)
        for d in range(D):
            ivd = ivecT[d, sl]
            acc_o = acc_o + ivd * ovecT[d, sl]
            acc_n = acc_n + ivd * nvecT[d, sl]
        od_v[sl] = acc_o
        nd_v[sl] = acc_n
        return carry

    lax.fori_loop(0, NBLK, blk_body, 0)

    pltpu.sync_copy(od_v, od_hbm.at[pl.ds(base, BPW)])
    pltpu.sync_copy(nd_v, nd_hbm.at[pl.ds(base, BPW)])


def _loss_body(od_ref, nd_ref, out_ref):
    od = od_ref[...]
    nd = nd_ref[...]
    lso = jnp.minimum(od, 0.0) - jnp.log1p(jnp.exp(-jnp.abs(od)))
    x = -nd
    lsn = jnp.minimum(x, 0.0) - jnp.log1p(jnp.exp(-jnp.abs(x)))
    out_ref[0, 0] = -(jnp.sum(lso) + jnp.sum(lsn)) / B


_tc_loss = pl.pallas_call(
    _loss_body,
    out_shape=jax.ShapeDtypeStruct((1, 1), jnp.float32),
    out_specs=pl.BlockSpec(memory_space=pltpu.SMEM),
)


def kernel(iword, owords, nwords, emb_i, emb_o):
    iword = iword.astype(jnp.int32)
    owords = owords.astype(jnp.int32)
    nwords = nwords.astype(jnp.int32)
    od, nd = _sc_dots(iword, owords, nwords, emb_i.T, emb_o.T)
    out = _tc_loss(od.reshape(128, 128), nd.reshape(128, 128))
    return out[0, 0]


# speedup vs baseline: 5.5831x; 5.5831x over previous
import functools

import jax
import jax.numpy as jnp
from jax import lax
from jax.experimental import pallas as pl
from jax.experimental.pallas import tpu as pltpu
from jax.experimental.pallas import tpu_sc as plsc

V = 1000000
D = 32
B = 16384
R = 128
VR = V * D // R

NC = 2
NS = 16
L = 16
NW = NC * NS
BPW = B // NW
CHUNK = 128
NCHUNK = BPW // CHUNK
NIDXBLK = BPW // L

_mesh = plsc.VectorSubcoreMesh(core_axis_name="c", subcore_axis_name="s")


@functools.partial(
    pl.kernel,
    mesh=_mesh,
    out_type=(
        jax.ShapeDtypeStruct((B,), jnp.float32),
        jax.ShapeDtypeStruct((B,), jnp.float32),
    ),
    scratch_types=[
        pltpu.VMEM((BPW,), jnp.int32),
        pltpu.VMEM((BPW,), jnp.int32),
        pltpu.VMEM((BPW,), jnp.int32),
        pltpu.VMEM((BPW,), jnp.int32),
        pltpu.VMEM((BPW,), jnp.int32),
        pltpu.VMEM((BPW,), jnp.int32),
        pltpu.VMEM((CHUNK, R), jnp.float32),
        pltpu.VMEM((CHUNK, R), jnp.float32),
        pltpu.VMEM((CHUNK, R), jnp.float32),
        pltpu.VMEM((BPW,), jnp.float32),
        pltpu.VMEM((BPW,), jnp.float32),
        pltpu.SemaphoreType.DMA,
    ],
    compiler_params=pltpu.CompilerParams(
        needs_layout_passes=False, use_tc_tiling_on_sc=False
    ),
)
def _sc_dots(iword, owords, nwords, emb_i4, emb_o4, od_hbm, nd_hbm,
             iraw, oraw, nraw, iidx, oidx, nidx,
             dsti, dsto, dstn, od_v, nd_v, sem):
    wid = lax.axis_index("s") * NC + lax.axis_index("c")
    base = wid * BPW

    pltpu.sync_copy(iword.at[pl.ds(base, BPW)], iraw)
    pltpu.sync_copy(owords.at[pl.ds(base, BPW)], oraw)
    pltpu.sync_copy(nwords.at[pl.ds(base, BPW)], nraw)

    def shift_body(b, carry):
        sl = pl.ds(b * L, L)
        iidx[sl] = jax.lax.shift_right_logical(iraw[sl], 2)
        oidx[sl] = jax.lax.shift_right_logical(oraw[sl], 2)
        nidx[sl] = jax.lax.shift_right_logical(nraw[sl], 2)
        return carry

    lax.fori_loop(0, NIDXBLK, shift_body, 0)

    lanes = lax.broadcasted_iota(jnp.int32, (L,), 0)

    def chunk_body(c, carry):
        csl = pl.ds(c * CHUNK, CHUNK)
        c1 = pltpu.async_copy(emb_i4.at[iidx.at[csl]], dsti, sem)
        c2 = pltpu.async_copy(emb_o4.at[oidx.at[csl]], dsto, sem)
        c3 = pltpu.async_copy(emb_o4.at[nidx.at[csl]], dstn, sem)
        c1.wait()
        c2.wait()
        c3.wait()

        def blk_body(b, carry2):
            jvec = b * L + lanes
            rsl = pl.ds(c * CHUNK + b * L, L)
            icol0 = (iraw[rsl] & 3) * D
            ocol0 = (oraw[rsl] & 3) * D
            ncol0 = (nraw[rsl] & 3) * D
            acc_o = jnp.zeros((L,), jnp.float32)
            acc_n = jnp.zeros((L,), jnp.float32)
            for d in range(D):
                iv = plsc.load_gather(dsti, [jvec, icol0 + d])
                ov = plsc.load_gather(dsto, [jvec, ocol0 + d])
                nv = plsc.load_gather(dstn, [jvec, ncol0 + d])
                acc_o = acc_o + iv * ov
                acc_n = acc_n + iv * nv
            od_v[rsl] = acc_o
            nd_v[rsl] = acc_n
            return carry2

        lax.fori_loop(0, CHUNK // L, blk_body, 0)
        return carry

    lax.fori_loop(0, NCHUNK, chunk_body, 0)

    pltpu.sync_copy(od_v, od_hbm.at[pl.ds(base, BPW)])
    pltpu.sync_copy(nd_v, nd_hbm.at[pl.ds(base, BPW)])


def _loss_body(od_ref, nd_ref, out_ref):
    od = od_ref[...]
    nd = nd_ref[...]
    lso = jnp.minimum(od, 0.0) - jnp.log1p(jnp.exp(-jnp.abs(od)))
    x = -nd
    lsn = jnp.minimum(x, 0.0) - jnp.log1p(jnp.exp(-jnp.abs(x)))
    out_ref[0, 0] = -(jnp.sum(lso) + jnp.sum(lsn)) / B


_tc_loss = pl.pallas_call(
    _loss_body,
    out_shape=jax.ShapeDtypeStruct((1, 1), jnp.float32),
    out_specs=pl.BlockSpec(memory_space=pltpu.SMEM),
)


def kernel(iword, owords, nwords, emb_i, emb_o):
    iword = iword.astype(jnp.int32)
    owords = owords.astype(jnp.int32)
    nwords = nwords.astype(jnp.int32)
    od, nd = _sc_dots(
        iword, owords, nwords,
        emb_i.reshape(VR, R), emb_o.reshape(VR, R),
    )
    out = _tc_loss(od.reshape(128, 128), nd.reshape(128, 128))
    return out[0, 0]


# speedup vs baseline: 5.7644x; 1.0325x over previous
import functools

import jax
import jax.numpy as jnp
from jax import lax
from jax.experimental import pallas as pl
from jax.experimental.pallas import tpu as pltpu
from jax.experimental.pallas import tpu_sc as plsc

V = 1000000
D = 32
B = 16384

NC = 2
NS = 16
L = 16
NW = NC * NS
BPW = B // NW
CHUNK = 128
NCHUNK = BPW // CHUNK
NBLK = BPW // L

_mesh = plsc.VectorSubcoreMesh(core_axis_name="c", subcore_axis_name="s")


@functools.partial(
    pl.kernel,
    mesh=_mesh,
    out_type=jax.ShapeDtypeStruct((NW, L), jnp.float32),
    scratch_types=[
        pltpu.VMEM((BPW,), jnp.int32),
        pltpu.VMEM((BPW,), jnp.int32),
        pltpu.VMEM((BPW,), jnp.int32),
        pltpu.VMEM((BPW, D), jnp.float32),
        pltpu.VMEM((BPW, D), jnp.float32),
        pltpu.VMEM((BPW, D), jnp.float32),
        pltpu.VMEM((L,), jnp.float32),
        pltpu.SemaphoreType.DMA,
    ],
    compiler_params=pltpu.CompilerParams(
        needs_layout_passes=False, use_tc_tiling_on_sc=False
    ),
)
def _sc_dots(iword, owords, nwords, emb_i, emb_o, part_hbm,
             iidx, oidx, nidx, ivec, ovec, nvec, part_v, sem):
    wid = lax.axis_index("s") * NC + lax.axis_index("c")
    base = wid * BPW

    pltpu.sync_copy(iword.at[pl.ds(base, BPW)], iidx)
    pltpu.sync_copy(owords.at[pl.ds(base, BPW)], oidx)
    pltpu.sync_copy(nwords.at[pl.ds(base, BPW)], nidx)

    copies = []
    for j in range(NCHUNK):
        sl = pl.ds(j * CHUNK, CHUNK)
        copies.append(pltpu.async_copy(emb_i.at[iidx.at[sl]], ivec.at[sl], sem))
        copies.append(pltpu.async_copy(emb_o.at[oidx.at[sl]], ovec.at[sl], sem))
        copies.append(pltpu.async_copy(emb_o.at[nidx.at[sl]], nvec.at[sl], sem))
    for c in copies:
        c.wait()

    lanes = lax.broadcasted_iota(jnp.int32, (L,), 0)

    def _log_sigmoid(x):
        t = jnp.exp(-jnp.abs(x))
        s = t / (t + 2.0)
        s2 = s * s
        p = 1.0 + s2 * (1.0 / 3.0 + s2 * (1.0 / 5.0 + s2 * (1.0 / 7.0 + s2 * (1.0 / 9.0))))
        return jnp.minimum(x, 0.0) - 2.0 * s * p

    def blk_body(b, part):
        acc_o = jnp.zeros((L,), jnp.float32)
        acc_n = jnp.zeros((L,), jnp.float32)
        for k in range(L):
            r = b * L + k
            iv0 = ivec[r, pl.ds(0, L)]
            iv1 = ivec[r, pl.ds(L, L)]
            ov0 = ovec[r, pl.ds(0, L)]
            ov1 = ovec[r, pl.ds(L, L)]
            nv0 = nvec[r, pl.ds(0, L)]
            nv1 = nvec[r, pl.ds(L, L)]
            so = jnp.sum(iv0 * ov0 + iv1 * ov1)
            sn = jnp.sum(iv0 * nv0 + iv1 * nv1)
            acc_o = jnp.where(lanes == k, so, acc_o)
            acc_n = jnp.where(lanes == k, sn, acc_n)
        return part + _log_sigmoid(acc_o) + _log_sigmoid(-acc_n)

    part = lax.fori_loop(0, NBLK, blk_body, jnp.zeros((L,), jnp.float32))
    part_v[...] = part
    pltpu.sync_copy(part_v, part_hbm.at[wid])


def _loss_body(part_ref, out_ref):
    out_ref[0, 0] = -jnp.sum(part_ref[...]) / B


_tc_loss = pl.pallas_call(
    _loss_body,
    out_shape=jax.ShapeDtypeStruct((1, 1), jnp.float32),
    out_specs=pl.BlockSpec(memory_space=pltpu.SMEM),
)


def kernel(iword, owords, nwords, emb_i, emb_o):
    iword = iword.astype(jnp.int32)
    owords = owords.astype(jnp.int32)
    nwords = nwords.astype(jnp.int32)
    parts = _sc_dots(iword, owords, nwords, emb_i, emb_o)
    out = _tc_loss(parts)
    return out[0, 0]
